# baseline (speedup 1.0000x reference)
import functools

import jax
import jax.numpy as jnp
from jax import lax
from jax.experimental import pallas as pl
from jax.experimental.pallas import tpu as pltpu
from jax.experimental.pallas import tpu_sc as plsc

_N_CLS = 80
_NCH = 5 + _N_CLS
_STRIDE_CONST = 32.0
_NTILE = 192
_CELLS = 256
_TILE_F32 = _NCH * _CELLS
_NWORK = 32
_TPW = _NTILE // _NWORK
_SUBS = ((0, 5504), (5504, 5504), (11008, 5504), (16512, 5248))


def _sub_copies(hbm, tile, buf, sem):
    for off, size in _SUBS:
        sl = pl.ds(off, size)
        yield hbm.at[tile, sl], buf.at[sl], sem


def _sc_decode_body(x_hbm, mm_hbm, bs_hbm, out_hbm,
                    in_a, in_b, out_a, out_b, mm_v, bs_v,
                    sem_ia, sem_ib, sem_oa, sem_ob):
    wid = lax.axis_index("s") * 2 + lax.axis_index("c")
    pltpu.sync_copy(mm_hbm, mm_v)
    pltpu.sync_copy(bs_hbm, bs_v)
    bx = bs_v[pl.ds(0, 16)]
    sv = bs_v[pl.ds(16, 16)]
    lane = lax.iota(jnp.int32, 16)
    ins = (in_a, in_b)
    outs = (out_a, out_b)
    sem_i = (sem_ia, sem_ib)
    sem_o = (sem_oa, sem_ob)

    def start_in(t):
        for src, dst, sem in _sub_copies(x_hbm, wid * _TPW + t, ins[t % 2],
                                         sem_i[t % 2]):
            pltpu.async_copy(src, dst, sem)

    def wait_in(t):
        for src, dst, sem in _sub_copies(x_hbm, wid * _TPW + t, ins[t % 2],
                                         sem_i[t % 2]):
            pltpu.make_async_copy(src, dst, sem).wait()

    def start_out(t):
        for src, dst, sem in _sub_copies(out_hbm, wid * _TPW + t,
                                         outs[t % 2], sem_o[t % 2]):
            pltpu.async_copy(dst, src, sem)

    def wait_out(t):
        for src, dst, sem in _sub_copies(out_hbm, wid * _TPW + t,
                                         outs[t % 2], sem_o[t % 2]):
            pltpu.make_async_copy(dst, src, sem).wait()

    start_in(0)
    for t in range(_TPW):
        tile = wid * _TPW + t
        in_v = ins[t % 2]
        out_v = outs[t % 2]
        wait_in(t)
        if t + 1 < _TPW:
            start_in(t + 1)
        if t >= 2:
            wait_out(t - 2)
        mmo = pl.multiple_of(tile * 32, 32)
        m1 = mm_v[pl.ds(mmo, 16)]
        m2 = mm_v[pl.ds(mmo + 16, 16)]

        @plsc.parallel_loop(4, _NCH)
        def _sig_loop(c):
            base = c * 256
            for k in range(16):
                sl = pl.ds(base + 16 * k, 16)
                v = in_v[sl]
                in_v[sl] = 1.0 / (1.0 + jnp.exp(-v))

        def _scat_loop(k, carry):
            g0 = 16 * k
            cidx = (g0 + lane) * _NCH
            el = jnp.exp(in_v[pl.ds(g0, 16)])
            et = jnp.exp(in_v[pl.ds(256 + g0, 16)])
            er = jnp.exp(in_v[pl.ds(512 + g0, 16)])
            eb = jnp.exp(in_v[pl.ds(768 + g0, 16)])
            ky = k.astype(jnp.float32) + 0.5
            xq = bx + (er - el) * m1
            yq = ky * sv + (eb - et) * m1
            wq = (el + er) * m2
            hq = (et + eb) * m2
            plsc.store_scatter(out_v, [cidx], xq)
            plsc.store_scatter(out_v, [cidx + 1], yq)
            plsc.store_scatter(out_v, [cidx + 2], wq)
            plsc.store_scatter(out_v, [cidx + 3], hq)
            for c in range(4, _NCH):
                plsc.store_scatter(out_v, [cidx + c],
                                   in_v[pl.ds(c * 256 + g0, 16)])
            return carry

        lax.fori_loop(0, 16, _scat_loop, 0)
        start_out(t)
    wait_out(_TPW - 2)
    wait_out(_TPW - 1)


_sc_decode = functools.partial(
    pl.kernel,
    out_type=jax.ShapeDtypeStruct((_NTILE, _TILE_F32), jnp.float32),
    mesh=plsc.VectorSubcoreMesh(core_axis_name="c", subcore_axis_name="s"),
    compiler_params=pltpu.CompilerParams(needs_layout_passes=False),
    scratch_types=[
        pltpu.VMEM((_TILE_F32,), jnp.float32),
        pltpu.VMEM((_TILE_F32,), jnp.float32),
        pltpu.VMEM((_TILE_F32,), jnp.float32),
        pltpu.VMEM((_TILE_F32,), jnp.float32),
        pltpu.VMEM((_NTILE * 32,), jnp.float32),
        pltpu.VMEM((32,), jnp.float32),
        pltpu.SemaphoreType.DMA,
        pltpu.SemaphoreType.DMA,
        pltpu.SemaphoreType.DMA,
        pltpu.SemaphoreType.DMA,
    ],
)(_sc_decode_body)


def kernel(raw, anchors, img_size):
    nB = raw.shape[0]
    nG = raw.shape[2]
    x = raw.reshape(_NTILE, _TILE_F32)
    s = jnp.asarray(img_size, jnp.float32) / nG
    aw_t = jnp.tile(anchors[:, 0], nB)
    ones16 = jnp.ones((1, 16), jnp.float32)
    m1 = (aw_t * (s / (2.0 * _STRIDE_CONST)))[:, None] * ones16
    m2 = (aw_t * (s / _STRIDE_CONST))[:, None] * ones16
    mm = jnp.concatenate([m1, m2], axis=1).reshape(_NTILE * 32)
    bx = (jnp.arange(16, dtype=jnp.float32) + 0.5) * s
    sv = jnp.full((16,), s, jnp.float32)
    bs = jnp.concatenate([bx, sv]).reshape(32)
    out = _sc_decode(x, mm, bs)
    return out.reshape(nB, _NTILE // nB * _CELLS, _NCH)


# speedup vs baseline: 2.6479x; 2.6479x over previous
import jax
import jax.numpy as jnp
from jax.experimental import pallas as pl
from jax.experimental.pallas import tpu as pltpu

_NCH = 85
_MB = 4
_NCHUNK = 16
_NSLOT = 3


def _decode_tiles(in_buf, out_buf, aw_ref, s):
    g = jax.lax.broadcasted_iota(jnp.int32, (1, 256), 1)
    gx = (g % 16).astype(jnp.float32)
    gy = (g // 16).astype(jnp.float32)
    half = s / 64.0
    bx = (gx + 0.5) * s
    by = (gy + 0.5) * s
    for m in range(_MB):
        for a in range(3):
            aw = aw_ref[a]
            x = in_buf[m, a]
            e = jnp.exp(x[0:4, :]) * aw
            l = e[0:1, :]
            t = e[1:2, :]
            r = e[2:3, :]
            b = e[3:4, :]
            xq = bx + (r - l) * half
            yq = by + (b - t) * half
            wq = (l + r) * (s / 32.0)
            hq = (t + b) * (s / 32.0)
            sig = jax.nn.sigmoid(x[4:_NCH, :])
            dec = jnp.concatenate([xq, yq, wq, hq, sig], axis=0)
            out_buf[m, a] = dec.T


def _ring_body(x_hbm, aw_ref, s_ref, o_hbm, in_bufs, out_bufs, sem_i, sem_o):
    s = s_ref[0]

    def in_copy(c, slot):
        return pltpu.make_async_copy(x_hbm.at[c], in_bufs.at[slot], sem_i.at[slot])

    def out_copy(c, slot):
        return pltpu.make_async_copy(out_bufs.at[slot], o_hbm.at[c], sem_o.at[slot])

    for c in range(_NSLOT):
        in_copy(c, c).start()
    for c in range(_NCHUNK):
        slot = c % _NSLOT
        in_copy(c, slot).wait()
        if c >= _NSLOT:
            out_copy(c - _NSLOT, slot).wait()
        _decode_tiles(in_bufs[slot], out_bufs.at[slot], aw_ref, s)
        out_copy(c, slot).start()
        nxt = c + _NSLOT
        if nxt < _NCHUNK:
            in_copy(nxt, slot).start()
    for c in range(_NCHUNK - _NSLOT, _NCHUNK):
        out_copy(c, c % _NSLOT).wait()


def kernel(raw, anchors, img_size):
    nB = raw.shape[0]
    nG = raw.shape[2]
    nA = anchors.shape[0]
    x = raw.reshape(_NCHUNK, _MB, nA, _NCH, nG * nG)
    stride = (jnp.asarray(img_size, jnp.float32) / nG).reshape(1)
    aw = anchors[:, 0]
    out = pl.pallas_call(
        _ring_body,
        in_specs=[
            pl.BlockSpec(memory_space=pl.ANY),
            pl.BlockSpec(memory_space=pltpu.SMEM),
            pl.BlockSpec(memory_space=pltpu.SMEM),
        ],
        out_specs=pl.BlockSpec(memory_space=pl.ANY),
        out_shape=jax.ShapeDtypeStruct((_NCHUNK, _MB, nA, nG * nG, _NCH),
                                       jnp.float32),
        scratch_shapes=[
            pltpu.VMEM((_NSLOT, _MB, nA, _NCH, nG * nG), jnp.float32),
            pltpu.VMEM((_NSLOT, _MB, nA, nG * nG, _NCH), jnp.float32),
            pltpu.SemaphoreType.DMA((_NSLOT,)),
            pltpu.SemaphoreType.DMA((_NSLOT,)),
        ],
    )(x, aw, stride)
    return out.reshape(nB, nA * nG * nG, _NCH)


# speedup vs baseline: 2.6716x; 1.0089x over previous
import jax
import jax.numpy as jnp
from jax.experimental import pallas as pl
from jax.experimental.pallas import tpu as pltpu

_N_CLS = 80
_NCH = 5 + _N_CLS
_STRIDE_CONST = 32.0
_MB = 32


def _decode_body(x_ref, aw_ref, s_ref, o_ref):
    s = s_ref[0]
    g = jax.lax.broadcasted_iota(jnp.int32, (1, 256), 1)
    gx = (g % 16).astype(jnp.float32)
    gy = (g // 16).astype(jnp.float32)
    half = s / (2.0 * _STRIDE_CONST)
    bx = (gx + 0.5) * s
    by = (gy + 0.5) * s
    for m in range(_MB):
        for a in range(3):
            aw = aw_ref[a]
            x = x_ref[m, a]
            e = jnp.exp(x[0:4, :]) * aw
            l = e[0:1, :]
            t = e[1:2, :]
            r = e[2:3, :]
            b = e[3:4, :]
            xq = bx + (r - l) * half
            yq = by + (b - t) * half
            wq = (l + r) * (s / _STRIDE_CONST)
            hq = (t + b) * (s / _STRIDE_CONST)
            sig = jax.nn.sigmoid(x[4:_NCH, :])
            dec = jnp.concatenate([xq, yq, wq, hq, sig], axis=0)
            o_ref[m, a] = dec.T


def kernel(raw, anchors, img_size):
    nB = raw.shape[0]
    nG = raw.shape[2]
    nA = anchors.shape[0]
    x = raw.reshape(nB, nA, _NCH, nG * nG)
    stride = (jnp.asarray(img_size, jnp.float32) / nG).reshape(1)
    aw = anchors[:, 0]

    out = pl.pallas_call(
        _decode_body,
        grid=(nB // _MB,),
        in_specs=[
            pl.BlockSpec((_MB, nA, _NCH, nG * nG), lambda i: (i, 0, 0, 0)),
            pl.BlockSpec(memory_space=pltpu.SMEM),
            pl.BlockSpec(memory_space=pltpu.SMEM),
        ],
        out_specs=pl.BlockSpec((_MB, nA, nG * nG, _NCH), lambda i: (i, 0, 0, 0)),
        out_shape=jax.ShapeDtypeStruct((nB, nA, nG * nG, _NCH), jnp.float32),
    )(x, aw, stride)
    return out.reshape(nB, nA * nG * nG, _NCH)
